# baseline (speedup 1.0000x reference)
import functools
import math

import jax
import jax.numpy as jnp
from jax import lax
from jax.experimental import pallas as pl
from jax.experimental.pallas import tpu as pltpu
from jax.experimental.pallas import tpu_sc as plsc

_BASE_RES = 16
_MAX_RES = 256
_NUM_LOD = 8
_FEAT = 2
_N = 1048576
_GROWTH = math.exp((math.log(_MAX_RES) - math.log(_BASE_RES)) / (_NUM_LOD - 1))
_LODS = [int(_BASE_RES * _GROWTH ** L) for L in range(_NUM_LOD)]

_SIZES_P = [-(-r * r // 8) * 8 for r in _LODS]
_PREF = [0]
for _s in _SIZES_P:
    _PREF.append(_PREF[-1] + _s)
_F8 = _PREF[-1]
_S05 = _PREF[6]

_TB = [[_PREF[l] + f * _S05 for l in range(6)] for f in range(2)]
_L6, _L7 = _SIZES_P[6], _SIZES_P[7]
_SB = {(0, 6): 0, (1, 6): _L6, (0, 7): 2 * _L6, (1, 7): 2 * _L6 + _L7}
_SPM = 2 * (_L6 + _L7)

_NC = 2
_NS = 16
_NW = _NC * _NS
_PPW = _N // _NW
_C = 1024
_CHUNKS = _PPW // _C
_PAIRS = _CHUNKS // 2


def _make_lookup():
    mesh = plsc.VectorSubcoreMesh(
        core_axis_name="c", subcore_axis_name="s",
        num_cores=_NC, num_subcores=_NS)

    @functools.partial(
        pl.kernel,
        out_type=jax.ShapeDtypeStruct((_N, _NUM_LOD * _FEAT), jnp.float32),
        mesh=mesh,
        compiler_params=pltpu.CompilerParams(
            needs_layout_passes=False, use_tc_tiling_on_sc=False),
        scratch_types=[
            pltpu.VMEM((2 * _S05,), jnp.float32),
            pltpu.VMEM((_C,), jnp.float32),
            pltpu.VMEM((_C,), jnp.float32),
            pltpu.VMEM((4 * _C,), jnp.int32),
            pltpu.VMEM((4 * _C,), jnp.float32),
            pltpu.VMEM((_C, 16), jnp.float32),
            pltpu.VMEM((4 * _C,), jnp.int32),
            pltpu.VMEM((4 * _C,), jnp.float32),
            pltpu.VMEM((_C, 16), jnp.float32),
            pltpu.VMEM_SHARED((_SPM,), jnp.float32),
            pltpu.SemaphoreType.DMA,
            pltpu.SemaphoreType.DMA,
        ],
    )
    def lookup(xs_h, ys_h, cbcat_h, out_h,
               tabv, xv, yv, idxA, colA, outA, idxB, colB, outB,
               spm, semA, semB):
        sid = lax.axis_index("s")
        wid = sid * _NC + lax.axis_index("c")
        iota = lax.iota(jnp.int32, 16)
        zeros16 = iota * 0
        ones16 = zeros16 + 1
        oc = [zeros16 + j for j in range(16)]

        pltpu.sync_copy(cbcat_h.at[pl.ds(0, _S05)], tabv.at[pl.ds(0, _S05)])
        pltpu.sync_copy(cbcat_h.at[pl.ds(_F8, _S05)],
                        tabv.at[pl.ds(_S05, _S05)])

        @pl.when(sid == 0)
        def _():
            for (f, l), b in _SB.items():
                pltpu.sync_copy(
                    cbcat_h.at[pl.ds(f * _F8 + _PREF[l], _SIZES_P[l])],
                    spm.at[pl.ds(b, _SIZES_P[l])])
        plsc.subcore_barrier()

        def compute(ci, idxv, colv, outv, sem):
            base = pl.multiple_of(wid * _PPW + ci * _C, _C)
            pltpu.sync_copy(xs_h.at[pl.ds(base, _C)], xv)
            pltpu.sync_copy(ys_h.at[pl.ds(base, _C)], yv)

            def idx_body(j, c2):
                rows = iota + j * 16
                x = xv[pl.ds(j * 16, 16)]
                y = yv[pl.ds(j * 16, 16)]
                for l in range(6):
                    r = _LODS[l]
                    cell = ((x * (r - 1.0)).astype(jnp.int32)
                            + (y * (r - 1.0)).astype(jnp.int32) * r)
                    f0 = plsc.load_gather(tabv, [cell + _TB[0][l]])
                    plsc.store_scatter(outv, [rows, oc[l]], f0)
                    f1 = plsc.load_gather(tabv, [cell + _TB[1][l]])
                    plsc.store_scatter(outv, [rows, oc[8 + l]], f1)
                for li, l in enumerate((6, 7)):
                    r = _LODS[l]
                    cell = ((x * (r - 1.0)).astype(jnp.int32)
                            + (y * (r - 1.0)).astype(jnp.int32) * r)
                    idxv[pl.ds((2 * li) * _C + j * 16, 16)] = (
                        cell + _SB[(0, l)])
                    idxv[pl.ds((2 * li + 1) * _C + j * 16, 16)] = (
                        cell + _SB[(1, l)])
                return c2
            lax.fori_loop(0, _C // 16, idx_body, 0)

            return pltpu.async_copy(spm.at[idxv], colv, sem)

        def finish(ci, cpy, colv, outv):
            cpy.wait()

            def asm_body(i, c2):
                rows = iota + i * 16
                for ki, j in enumerate((6, 14, 7, 15)):
                    v = colv[pl.ds(ki * _C + i * 16, 16)]
                    plsc.store_scatter(outv, [rows, oc[j]], v)
                return c2
            lax.fori_loop(0, _C // 16, asm_body, 0)

            base = pl.multiple_of(wid * _PPW + ci * _C, _C)
            pltpu.sync_copy(outv, out_h.at[pl.ds(base, _C), :])

        def pair_body(p, carry):
            a = p * 2
            b = a + 1
            cA = compute(a, idxA, colA, outA, semA)
            cB = compute(b, idxB, colB, outB, semB)
            finish(a, cA, colA, outA)
            finish(b, cB, colB, outB)
            return carry

        lax.fori_loop(0, _PAIRS, pair_body, 0)

    return lookup


_lookup = _make_lookup()


def kernel(pts, cb0, cb1, cb2, cb3, cb4, cb5, cb6, cb7):
    cbs = [cb0, cb1, cb2, cb3, cb4, cb5, cb6, cb7]
    pieces = []
    for f in range(2):
        for l, cb in enumerate(cbs):
            col = jnp.ravel(cb[:, f])
            pad = _SIZES_P[l] - col.shape[0]
            if pad:
                col = jnp.concatenate([col, jnp.zeros((pad,), jnp.float32)])
            pieces.append(col)
    cbcat = jnp.concatenate(pieces)
    ptsT = pts.T
    return _lookup(ptsT[0], ptsT[1], cbcat)


# speedup vs baseline: 1.0050x; 1.0050x over previous
import functools
import math

import jax
import jax.numpy as jnp
from jax import lax
from jax.experimental import pallas as pl
from jax.experimental.pallas import tpu as pltpu
from jax.experimental.pallas import tpu_sc as plsc

_BASE_RES = 16
_MAX_RES = 256
_NUM_LOD = 8
_FEAT = 2
_N = 1048576
_GROWTH = math.exp((math.log(_MAX_RES) - math.log(_BASE_RES)) / (_NUM_LOD - 1))
_LODS = [int(_BASE_RES * _GROWTH ** L) for L in range(_NUM_LOD)]

_SIZES_P = [-(-r * r // 8) * 8 for r in _LODS]
_PREF = [0]
for _s in _SIZES_P:
    _PREF.append(_PREF[-1] + _s)
_F8 = _PREF[-1]
_S05 = _PREF[6]

_TB = [[_PREF[l] + f * _S05 for l in range(6)] for f in range(2)]
_L6, _L7 = _SIZES_P[6], _SIZES_P[7]
_SB = {(0, 6): 0, (1, 6): _L6, (0, 7): 2 * _L6, (1, 7): 2 * _L6 + _L7}
_SPM = 2 * (_L6 + _L7)

_NC = 2
_NS = 16
_NW = _NC * _NS
_PPW = _N // _NW
_C = 1024
_CHUNKS = _PPW // _C
_PAIRS = _CHUNKS // 2


def _make_lookup():
    mesh = plsc.VectorSubcoreMesh(
        core_axis_name="c", subcore_axis_name="s",
        num_cores=_NC, num_subcores=_NS)

    @functools.partial(
        pl.kernel,
        out_type=jax.ShapeDtypeStruct((_N, _NUM_LOD * _FEAT), jnp.float32),
        mesh=mesh,
        compiler_params=pltpu.CompilerParams(
            needs_layout_passes=False, use_tc_tiling_on_sc=False),
        scratch_types=[
            pltpu.VMEM((2 * _S05,), jnp.float32),
            pltpu.VMEM((_C,), jnp.float32),
            pltpu.VMEM((_C,), jnp.float32),
            pltpu.VMEM((4 * _C,), jnp.int32),
            pltpu.VMEM((4 * _C,), jnp.float32),
            pltpu.VMEM((_C, 16), jnp.float32),
            pltpu.VMEM((4 * _C,), jnp.int32),
            pltpu.VMEM((4 * _C,), jnp.float32),
            pltpu.VMEM((_C, 16), jnp.float32),
            pltpu.VMEM_SHARED((_SPM,), jnp.float32),
            pltpu.SemaphoreType.DMA,
            pltpu.SemaphoreType.DMA,
        ],
    )
    def lookup(xs_h, ys_h, cbcat_h, out_h,
               tabv, xv, yv, idxA, colA, outA, idxB, colB, outB,
               spm, semA, semB):
        sid = lax.axis_index("s")
        wid = sid * _NC + lax.axis_index("c")
        iota = lax.iota(jnp.int32, 16)
        zeros16 = iota * 0
        ones16 = zeros16 + 1
        oc = [zeros16 + j for j in range(16)]

        pltpu.sync_copy(cbcat_h.at[pl.ds(0, _S05)], tabv.at[pl.ds(0, _S05)])
        pltpu.sync_copy(cbcat_h.at[pl.ds(_F8, _S05)],
                        tabv.at[pl.ds(_S05, _S05)])

        @pl.when(sid == 0)
        def _():
            for (f, l), b in _SB.items():
                pltpu.sync_copy(
                    cbcat_h.at[pl.ds(f * _F8 + _PREF[l], _SIZES_P[l])],
                    spm.at[pl.ds(b, _SIZES_P[l])])
        plsc.subcore_barrier()

        def compute(ci, idxv, colv, outv, sem):
            base = pl.multiple_of(wid * _PPW + ci * _C, _C)
            pltpu.sync_copy(xs_h.at[pl.ds(base, _C)], xv)
            pltpu.sync_copy(ys_h.at[pl.ds(base, _C)], yv)

            def idx_body(j, c2):
                rows = iota + j * 16
                x = xv[pl.ds(j * 16, 16)]
                y = yv[pl.ds(j * 16, 16)]
                for l in range(6):
                    r = _LODS[l]
                    cell = ((x * (r - 1.0)).astype(jnp.int32)
                            + (y * (r - 1.0)).astype(jnp.int32) * r)
                    f0 = plsc.load_gather(tabv, [cell + _TB[0][l]])
                    plsc.store_scatter(outv, [rows, oc[l]], f0)
                    f1 = plsc.load_gather(tabv, [cell + _TB[1][l]])
                    plsc.store_scatter(outv, [rows, oc[8 + l]], f1)
                for li, l in enumerate((6, 7)):
                    r = _LODS[l]
                    cell = ((x * (r - 1.0)).astype(jnp.int32)
                            + (y * (r - 1.0)).astype(jnp.int32) * r)
                    idxv[pl.ds((2 * li) * _C + j * 16, 16)] = (
                        cell + _SB[(0, l)])
                    idxv[pl.ds((2 * li + 1) * _C + j * 16, 16)] = (
                        cell + _SB[(1, l)])
                return c2
            lax.fori_loop(0, _C // 16, idx_body, 0, unroll=2)

            return pltpu.async_copy(spm.at[idxv], colv, sem)

        def finish(ci, cpy, colv, outv):
            cpy.wait()

            def asm_body(i, c2):
                rows = iota + i * 16
                for ki, j in enumerate((6, 14, 7, 15)):
                    v = colv[pl.ds(ki * _C + i * 16, 16)]
                    plsc.store_scatter(outv, [rows, oc[j]], v)
                return c2
            lax.fori_loop(0, _C // 16, asm_body, 0, unroll=4)

            base = pl.multiple_of(wid * _PPW + ci * _C, _C)
            pltpu.sync_copy(outv, out_h.at[pl.ds(base, _C), :])

        def pair_body(p, carry):
            a = p * 2
            b = a + 1
            cA = compute(a, idxA, colA, outA, semA)
            cB = compute(b, idxB, colB, outB, semB)
            finish(a, cA, colA, outA)
            finish(b, cB, colB, outB)
            return carry

        lax.fori_loop(0, _PAIRS, pair_body, 0)

    return lookup


_lookup = _make_lookup()


def kernel(pts, cb0, cb1, cb2, cb3, cb4, cb5, cb6, cb7):
    cbs = [cb0, cb1, cb2, cb3, cb4, cb5, cb6, cb7]
    pieces = []
    for f in range(2):
        for l, cb in enumerate(cbs):
            col = jnp.ravel(cb[:, f])
            pad = _SIZES_P[l] - col.shape[0]
            if pad:
                col = jnp.concatenate([col, jnp.zeros((pad,), jnp.float32)])
            pieces.append(col)
    cbcat = jnp.concatenate(pieces)
    ptsT = pts.T
    return _lookup(ptsT[0], ptsT[1], cbcat)
